# baseline (speedup 1.0000x reference)
import functools

import jax
import jax.numpy as jnp
from jax import lax
from jax.experimental import pallas as pl
from jax.experimental.pallas import tpu as pltpu
from jax.experimental.pallas import tpu_sc as plsc


def _sc_gather(N, D, C, n_per_w, n_chunks, num_cores):
    mesh = plsc.VectorSubcoreMesh(core_axis_name="c", subcore_axis_name="s")

    @functools.partial(
        pl.kernel,
        mesh=mesh,
        out_type=jax.ShapeDtypeStruct((N, D), jnp.float32),
        scratch_types=[
            pltpu.VMEM((C,), jnp.int32),
            pltpu.VMEM((C, D), jnp.float32),
            pltpu.SemaphoreType.DMA,
        ],
        compiler_params=pltpu.CompilerParams(use_tc_tiling_on_sc=False),
    )
    def k(table_hbm, idx_hbm, out_hbm, idx_v, rows_v, sem):
        wid = lax.axis_index("s") * num_cores + lax.axis_index("c")
        base = wid * n_per_w

        def body(c, carry):
            off = base + c * C
            pltpu.sync_copy(idx_hbm.at[pl.ds(off, C)], idx_v)
            pltpu.async_copy(table_hbm.at[idx_v], rows_v, sem).wait()
            pltpu.sync_copy(rows_v, out_hbm.at[pl.ds(off, C)])
            return carry

        lax.fori_loop(0, n_chunks, body, 0)

    return k


def kernel(x, table):
    B, S = x.shape
    V, D = table.shape
    N = B * S
    idx = x.reshape(N).astype(jnp.int32)

    info = plsc.get_sparse_core_info()
    NW = info.num_cores * info.num_subcores
    n_per_w = N // NW
    C = 800
    n_chunks = n_per_w // C

    out = _sc_gather(N, D, C, n_per_w, n_chunks, info.num_cores)(table, idx)
    return out.reshape(B, S, D)


# speedup vs baseline: 1.0225x; 1.0225x over previous
import functools

import jax
import jax.numpy as jnp
from jax import lax
from jax.experimental import pallas as pl
from jax.experimental.pallas import tpu as pltpu
from jax.experimental.pallas import tpu_sc as plsc

_C = 128
_K = 4


def _sc_gather(N, D, num_cores, num_subcores):
    NW = num_cores * num_subcores
    C, K = _C, _K
    n_chunks_w = N // (NW * C)
    n_super = n_chunks_w // K
    mesh = plsc.VectorSubcoreMesh(core_axis_name="c", subcore_axis_name="s")

    @functools.partial(
        pl.kernel,
        mesh=mesh,
        out_type=jax.ShapeDtypeStruct((N, D), jnp.float32),
        scratch_types=[
            pltpu.VMEM((n_chunks_w * C,), jnp.int32),
            pltpu.VMEM((2, K, C, D), jnp.float32),
            pltpu.SemaphoreType.DMA,
            pltpu.SemaphoreType.DMA,
            pltpu.SemaphoreType.DMA,
            pltpu.SemaphoreType.DMA,
        ],
        compiler_params=pltpu.CompilerParams(use_tc_tiling_on_sc=False),
    )
    def k(table_hbm, idx_hbm, out_hbm, idx_v, rows_v, g0, g1, s0, s1):
        gsem = (g0, g1)
        ssem = (s0, s1)
        wid = lax.axis_index("s") * num_cores + lax.axis_index("c")
        base = wid * n_chunks_w * C

        pltpu.sync_copy(idx_hbm.at[pl.ds(base, n_chunks_w * C)], idx_v)

        def gath(s, p, j, local):
            return pltpu.make_async_copy(
                table_hbm.at[idx_v.at[pl.ds(local * C, C)]],
                rows_v.at[p, j], gsem[p])

        def stor(s, p, j, local):
            off = base + local * C
            return pltpu.make_async_copy(
                rows_v.at[p, j], out_hbm.at[pl.ds(off, C)], ssem[p])

        def fire_gathers(s, p):
            for j in range(K):
                gath(s, p, j, s * K + j).start()

        def drain_gathers(s, p):
            for j in range(K):
                gath(s, p, j, s * K + j).wait()

        def fire_stores(s, p):
            for j in range(K):
                stor(s, p, j, s * K + j).start()

        def drain_stores(s, p):
            for j in range(K):
                stor(s, p, j, s * K + j).wait()

        fire_gathers(0, 0)
        drain_gathers(0, 0)
        fire_stores(0, 0)
        fire_gathers(1, 1)

        def body(i, carry):
            s1 = 2 * i + 1
            s2 = 2 * i + 2
            drain_gathers(s1, 1)
            fire_stores(s1, 1)
            drain_stores(s1 - 1, 0)
            fire_gathers(s1 + 1, 0)
            drain_gathers(s2, 0)
            fire_stores(s2, 0)
            drain_stores(s2 - 1, 1)
            fire_gathers(s2 + 1, 1)
            return carry

        lax.fori_loop(0, (n_super - 2) // 2, body, 0)

        s_last = n_super - 1
        drain_gathers(s_last, 1)
        fire_stores(s_last, 1)
        drain_stores(s_last - 1, 0)
        drain_stores(s_last, 1)

    return k


def kernel(x, table):
    B, S = x.shape
    V, D = table.shape
    N = B * S

    info = plsc.get_sparse_core_info()
    idx = x.reshape(N).astype(jnp.int32)
    out = _sc_gather(N, D, info.num_cores, info.num_subcores)(table, idx)
    return out.reshape(B, S, D)


# speedup vs baseline: 1.0248x; 1.0023x over previous
import functools

import jax
import jax.numpy as jnp
from jax import lax
from jax.experimental import pallas as pl
from jax.experimental.pallas import tpu as pltpu
from jax.experimental.pallas import tpu_sc as plsc

_NSLOT = 4


def _sc_embed(B, S, D, num_cores, num_subcores):
    NW = num_cores * num_subcores
    n_b = B // NW
    SP = 256
    mesh = plsc.VectorSubcoreMesh(core_axis_name="c", subcore_axis_name="s")

    @functools.partial(
        pl.kernel,
        mesh=mesh,
        out_type=jax.ShapeDtypeStruct((B, S, D), jnp.float32),
        scratch_types=[
            pltpu.VMEM((_NSLOT * SP,), jnp.int32),
            pltpu.VMEM((_NSLOT, S, D), jnp.float32),
        ]
        + [pltpu.SemaphoreType.DMA] * (3 * _NSLOT),
        compiler_params=pltpu.CompilerParams(use_tc_tiling_on_sc=False),
    )
    def k(table_hbm, x_hbm, out_hbm, idx_v, rows_v, *sems):
        isem = sems[0:_NSLOT]
        gsem = sems[_NSLOT:2 * _NSLOT]
        ssem = sems[2 * _NSLOT:3 * _NSLOT]
        wid = lax.axis_index("s") * num_cores + lax.axis_index("c")
        b0 = wid * n_b

        def idx_cp(i, r):
            return pltpu.make_async_copy(
                x_hbm.at[b0 + i], idx_v.at[pl.ds(r * SP, S)], isem[r])

        def gath_cp(i, r):
            return pltpu.make_async_copy(
                table_hbm.at[idx_v.at[pl.ds(r * SP, S)]], rows_v.at[r],
                gsem[r])

        def stor_cp(i, r):
            return pltpu.make_async_copy(
                rows_v.at[r], out_hbm.at[b0 + i], ssem[r])

        def body(i, r):
            stor_cp(i - 2, (r - 2) % _NSLOT).wait()
            idx_cp(i + 2, (r + 2) % _NSLOT).wait()
            gath_cp(i + 2, (r + 2) % _NSLOT).start()
            gath_cp(i, r).wait()
            stor_cp(i, r).start()
            idx_cp(i + 4, r).start()

        for i in range(_NSLOT):
            idx_cp(i, i).start()
        for i in range(2):
            idx_cp(i, i).wait()
            gath_cp(i, i).start()
        idx_cp(2, 2).wait()
        gath_cp(2, 2).start()
        gath_cp(0, 0).wait()
        stor_cp(0, 0).start()
        idx_cp(4, 0).start()
        idx_cp(3, 3).wait()
        gath_cp(3, 3).start()
        gath_cp(1, 1).wait()
        stor_cp(1, 1).start()
        idx_cp(5, 1).start()
        body(2, 2)
        body(3, 3)

        def outer(kk, carry):
            i4 = 4 * kk
            for r in range(_NSLOT):
                body(i4 + r, r)
            return carry

        lax.fori_loop(1, (n_b - 4) // 4, outer, 0)

        i = n_b - 4
        stor_cp(i - 2, (i - 2) % _NSLOT).wait()
        idx_cp(i + 2, (i + 2) % _NSLOT).wait()
        gath_cp(i + 2, (i + 2) % _NSLOT).start()
        gath_cp(i, i % _NSLOT).wait()
        stor_cp(i, i % _NSLOT).start()
        i = n_b - 3
        stor_cp(i - 2, (i - 2) % _NSLOT).wait()
        idx_cp(i + 2, (i + 2) % _NSLOT).wait()
        gath_cp(i + 2, (i + 2) % _NSLOT).start()
        gath_cp(i, i % _NSLOT).wait()
        stor_cp(i, i % _NSLOT).start()
        for i in range(n_b - 2, n_b):
            gath_cp(i, i % _NSLOT).wait()
            stor_cp(i, i % _NSLOT).start()
        for i in range(n_b - 4, n_b):
            stor_cp(i, i % _NSLOT).wait()

    return k


def kernel(x, table):
    B, S = x.shape
    V, D = table.shape
    info = plsc.get_sparse_core_info()
    xi = x.astype(jnp.int32)
    return _sc_embed(B, S, D, info.num_cores, info.num_subcores)(table, xi)


# speedup vs baseline: 1.3633x; 1.3303x over previous
import functools

import jax
import jax.numpy as jnp
from jax import lax
from jax.experimental import pallas as pl
from jax.experimental.pallas import tpu as pltpu
from jax.experimental.pallas import tpu_sc as plsc

_NSLOT = 4


def _sc_embed(B, S, D, num_cores, num_subcores):
    NW = num_cores * num_subcores
    n_b = B // NW
    SP = 256
    mesh = plsc.VectorSubcoreMesh(core_axis_name="c", subcore_axis_name="s")

    @functools.partial(
        pl.kernel,
        mesh=mesh,
        out_type=jax.ShapeDtypeStruct((B * S, 2 * D), jnp.float32),
        scratch_types=[
            pltpu.VMEM((_NSLOT * SP,), jnp.int32),
            pltpu.VMEM((_NSLOT, S, D), jnp.float32),
        ]
        + [pltpu.SemaphoreType.DMA] * (3 * _NSLOT),
        compiler_params=pltpu.CompilerParams(use_tc_tiling_on_sc=False),
    )
    def k(table_hbm, x_hbm, out_hbm, idx_v, rows_v, *sems):
        isem = sems[0:_NSLOT]
        gsem = sems[_NSLOT:2 * _NSLOT]
        ssem = sems[2 * _NSLOT:3 * _NSLOT]
        wid = lax.axis_index("s") * num_cores + lax.axis_index("c")
        b0 = wid * n_b

        def idx_cp(i, r):
            return pltpu.make_async_copy(
                x_hbm.at[b0 + i], idx_v.at[pl.ds(r * SP, S)], isem[r])

        def gath_cp(i, r):
            return pltpu.make_async_copy(
                table_hbm.at[idx_v.at[pl.ds(r * SP, S)]], rows_v.at[r],
                gsem[r])

        def stor_cp(i, r):
            return pltpu.make_async_copy(
                rows_v.at[r],
                out_hbm.at[pl.ds((b0 + i) * S, S), pl.ds(0, D)], ssem[r])

        def body(i, r):
            stor_cp(i - 2, (r - 2) % _NSLOT).wait()
            idx_cp(i + 2, (r + 2) % _NSLOT).wait()
            gath_cp(i + 2, (r + 2) % _NSLOT).start()
            gath_cp(i, r).wait()
            stor_cp(i, r).start()
            idx_cp(i + 4, r).start()

        for i in range(_NSLOT):
            idx_cp(i, i).start()
        for i in range(2):
            idx_cp(i, i).wait()
            gath_cp(i, i).start()
        idx_cp(2, 2).wait()
        gath_cp(2, 2).start()
        gath_cp(0, 0).wait()
        stor_cp(0, 0).start()
        idx_cp(4, 0).start()
        idx_cp(3, 3).wait()
        gath_cp(3, 3).start()
        gath_cp(1, 1).wait()
        stor_cp(1, 1).start()
        idx_cp(5, 1).start()
        body(2, 2)
        body(3, 3)

        def outer(kk, carry):
            i4 = 4 * kk
            for r in range(_NSLOT):
                body(i4 + r, r)
            return carry

        lax.fori_loop(1, (n_b - 4) // 4, outer, 0)

        i = n_b - 4
        stor_cp(i - 2, (i - 2) % _NSLOT).wait()
        idx_cp(i + 2, (i + 2) % _NSLOT).wait()
        gath_cp(i + 2, (i + 2) % _NSLOT).start()
        gath_cp(i, i % _NSLOT).wait()
        stor_cp(i, i % _NSLOT).start()
        i = n_b - 3
        stor_cp(i - 2, (i - 2) % _NSLOT).wait()
        idx_cp(i + 2, (i + 2) % _NSLOT).wait()
        gath_cp(i + 2, (i + 2) % _NSLOT).start()
        gath_cp(i, i % _NSLOT).wait()
        stor_cp(i, i % _NSLOT).start()
        for i in range(n_b - 2, n_b):
            gath_cp(i, i % _NSLOT).wait()
            stor_cp(i, i % _NSLOT).start()
        for i in range(n_b - 4, n_b):
            stor_cp(i, i % _NSLOT).wait()

    return k


def kernel(x, table):
    B, S = x.shape
    V, D = table.shape
    info = plsc.get_sparse_core_info()
    xi = x.astype(jnp.int32)
    out = _sc_embed(B, S, D, info.num_cores, info.num_subcores)(table, xi)
    return out[:, :D].reshape(B, S, D)


# speedup vs baseline: 1.5999x; 1.1736x over previous
import functools

import jax
import jax.numpy as jnp
from jax import lax
from jax.experimental import pallas as pl
from jax.experimental.pallas import tpu as pltpu
from jax.experimental.pallas import tpu_sc as plsc

_NSLOT = 4
_BV = 2048
_H = _BV // 2


def _tc_untile(V, D):
    grid = pl.cdiv(V, _BV)
    Vp = grid * _BV

    def body(i_ref, o_ref):
        x1 = i_ref[:, :_H].T
        x2 = i_ref[:, _H:].T
        o_ref[...] = jnp.concatenate([x1, x2], axis=1)

    return pl.pallas_call(
        body,
        grid=(grid,),
        in_specs=[pl.BlockSpec((D, _BV), lambda j: (0, j))],
        out_specs=pl.BlockSpec((_BV // 2, 2 * D), lambda j: (j, 0)),
        out_shape=jax.ShapeDtypeStruct((Vp * D // 128, 128), jnp.float32),
    ), Vp


def _sc_embed(B, S, D, num_cores, num_subcores):
    NW = num_cores * num_subcores
    n_b = B // NW
    SP = 256
    mesh = plsc.VectorSubcoreMesh(core_axis_name="c", subcore_axis_name="s")

    @functools.partial(
        pl.kernel,
        mesh=mesh,
        out_type=jax.ShapeDtypeStruct((B * S, 2 * D), jnp.float32),
        scratch_types=[
            pltpu.VMEM((_NSLOT * SP,), jnp.int32),
            pltpu.VMEM((_NSLOT, S, D), jnp.float32),
        ]
        + [pltpu.SemaphoreType.DMA] * (3 * _NSLOT),
        compiler_params=pltpu.CompilerParams(use_tc_tiling_on_sc=False),
    )
    def k(table_hbm, x_hbm, out_hbm, idx_v, rows_v, *sems):
        isem = sems[0:_NSLOT]
        gsem = sems[_NSLOT:2 * _NSLOT]
        ssem = sems[2 * _NSLOT:3 * _NSLOT]
        wid = lax.axis_index("s") * num_cores + lax.axis_index("c")
        b0 = wid * n_b

        def idx_cp(i, r):
            return pltpu.make_async_copy(
                x_hbm.at[b0 + i], idx_v.at[pl.ds(r * SP, S)], isem[r])

        def gath_cp(i, r):
            return pltpu.make_async_copy(
                table_hbm.at[idx_v.at[pl.ds(r * SP, S)]], rows_v.at[r],
                gsem[r])

        def stor_cp(i, r):
            return pltpu.make_async_copy(
                rows_v.at[r],
                out_hbm.at[pl.ds((b0 + i) * S, S), pl.ds(0, D)], ssem[r])

        def body(i, r):
            stor_cp(i - 2, (r - 2) % _NSLOT).wait()
            idx_cp(i + 2, (r + 2) % _NSLOT).wait()
            gath_cp(i + 2, (r + 2) % _NSLOT).start()
            gath_cp(i, r).wait()
            stor_cp(i, r).start()
            idx_cp(i + 4, r).start()

        for i in range(_NSLOT):
            idx_cp(i, i).start()
        for i in range(2):
            idx_cp(i, i).wait()
            gath_cp(i, i).start()
        idx_cp(2, 2).wait()
        gath_cp(2, 2).start()
        gath_cp(0, 0).wait()
        stor_cp(0, 0).start()
        idx_cp(4, 0).start()
        idx_cp(3, 3).wait()
        gath_cp(3, 3).start()
        gath_cp(1, 1).wait()
        stor_cp(1, 1).start()
        idx_cp(5, 1).start()
        body(2, 2)
        body(3, 3)

        def outer(kk, carry):
            i4 = 4 * kk
            for r in range(_NSLOT):
                body(i4 + r, r)
            return carry

        lax.fori_loop(1, (n_b - 4) // 4, outer, 0)

        i = n_b - 4
        stor_cp(i - 2, (i - 2) % _NSLOT).wait()
        idx_cp(i + 2, (i + 2) % _NSLOT).wait()
        gath_cp(i + 2, (i + 2) % _NSLOT).start()
        gath_cp(i, i % _NSLOT).wait()
        stor_cp(i, i % _NSLOT).start()
        i = n_b - 3
        stor_cp(i - 2, (i - 2) % _NSLOT).wait()
        idx_cp(i + 2, (i + 2) % _NSLOT).wait()
        gath_cp(i + 2, (i + 2) % _NSLOT).start()
        gath_cp(i, i % _NSLOT).wait()
        stor_cp(i, i % _NSLOT).start()
        for i in range(n_b - 2, n_b):
            gath_cp(i, i % _NSLOT).wait()
            stor_cp(i, i % _NSLOT).start()
        for i in range(n_b - 4, n_b):
            stor_cp(i, i % _NSLOT).wait()

    return k


def kernel(x, table):
    B, S = x.shape
    V, D = table.shape
    info = plsc.get_sparse_core_info()
    untile, Vp = _tc_untile(V, D)
    t_lin = untile(table.T).reshape(Vp, D)
    xi = x.astype(jnp.int32)
    hbits = _H.bit_length() - 1
    xi = (xi & ~(_BV - 1)) | ((xi & (_H - 1)) << 1) | ((xi >> hbits) & 1)
    out = _sc_embed(B, S, D, info.num_cores, info.num_subcores)(t_lin, xi)
    return out[:, :D].reshape(B, S, D)


# speedup vs baseline: 2.0971x; 1.3108x over previous
import functools

import jax
import jax.numpy as jnp
from jax import lax
from jax.experimental import pallas as pl
from jax.experimental.pallas import tpu as pltpu
from jax.experimental.pallas import tpu_sc as plsc

_NSLOT = 4
_BV = 8192
_H = _BV // 2


def _tc_untile(V, D):
    grid = pl.cdiv(V, _BV)
    Vp = grid * _BV

    def body(i_ref, o_ref):
        x1 = i_ref[:, :_H].T
        x2 = i_ref[:, _H:].T
        o_ref[...] = jnp.concatenate([x1, x2], axis=1)

    return pl.pallas_call(
        body,
        grid=(grid,),
        in_specs=[pl.BlockSpec((D, _BV), lambda j: (0, j))],
        out_specs=pl.BlockSpec((_BV // 2, 2 * D), lambda j: (j, 0)),
        out_shape=jax.ShapeDtypeStruct((Vp * D // 128, 128), jnp.float32),
    ), Vp


def _sc_embed(B, S, D, num_cores, num_subcores):
    NW = num_cores * num_subcores
    n_b = B // NW
    SP = 256
    mesh = plsc.VectorSubcoreMesh(core_axis_name="c", subcore_axis_name="s")

    @functools.partial(
        pl.kernel,
        mesh=mesh,
        out_type=jax.ShapeDtypeStruct((B * S, 2 * D), jnp.float32),
        scratch_types=[
            pltpu.VMEM((_NSLOT * SP,), jnp.int32),
            pltpu.VMEM((_NSLOT, S, D), jnp.float32),
        ]
        + [pltpu.SemaphoreType.DMA] * (3 * _NSLOT),
        compiler_params=pltpu.CompilerParams(use_tc_tiling_on_sc=False),
    )
    def k(table_hbm, x_hbm, out_hbm, idx_v, rows_v, *sems):
        isem = sems[0:_NSLOT]
        gsem = sems[_NSLOT:2 * _NSLOT]
        ssem = sems[2 * _NSLOT:3 * _NSLOT]
        wid = lax.axis_index("s") * num_cores + lax.axis_index("c")
        b0 = wid * n_b

        def idx_cp(i, r):
            return pltpu.make_async_copy(
                x_hbm.at[b0 + i], idx_v.at[pl.ds(r * SP, S)], isem[r])

        def gath_cp(i, r):
            return pltpu.make_async_copy(
                table_hbm.at[idx_v.at[pl.ds(r * SP, S)]], rows_v.at[r],
                gsem[r])

        def stor_cp(i, r):
            return pltpu.make_async_copy(
                rows_v.at[r],
                out_hbm.at[pl.ds((b0 + i) * S, S), pl.ds(0, D)], ssem[r])

        def body(i, r):
            stor_cp(i - 2, (r - 2) % _NSLOT).wait()
            idx_cp(i + 2, (r + 2) % _NSLOT).wait()
            gath_cp(i + 2, (r + 2) % _NSLOT).start()
            gath_cp(i, r).wait()
            stor_cp(i, r).start()
            idx_cp(i + 4, r).start()

        for i in range(_NSLOT):
            idx_cp(i, i).start()
        for i in range(2):
            idx_cp(i, i).wait()
            gath_cp(i, i).start()
        idx_cp(2, 2).wait()
        gath_cp(2, 2).start()
        gath_cp(0, 0).wait()
        stor_cp(0, 0).start()
        idx_cp(4, 0).start()
        idx_cp(3, 3).wait()
        gath_cp(3, 3).start()
        gath_cp(1, 1).wait()
        stor_cp(1, 1).start()
        idx_cp(5, 1).start()
        body(2, 2)
        body(3, 3)

        def outer(kk, carry):
            i4 = 4 * kk
            for r in range(_NSLOT):
                body(i4 + r, r)
            return carry

        lax.fori_loop(1, (n_b - 4) // 4, outer, 0)

        i = n_b - 4
        stor_cp(i - 2, (i - 2) % _NSLOT).wait()
        idx_cp(i + 2, (i + 2) % _NSLOT).wait()
        gath_cp(i + 2, (i + 2) % _NSLOT).start()
        gath_cp(i, i % _NSLOT).wait()
        stor_cp(i, i % _NSLOT).start()
        i = n_b - 3
        stor_cp(i - 2, (i - 2) % _NSLOT).wait()
        idx_cp(i + 2, (i + 2) % _NSLOT).wait()
        gath_cp(i + 2, (i + 2) % _NSLOT).start()
        gath_cp(i, i % _NSLOT).wait()
        stor_cp(i, i % _NSLOT).start()
        for i in range(n_b - 2, n_b):
            gath_cp(i, i % _NSLOT).wait()
            stor_cp(i, i % _NSLOT).start()
        for i in range(n_b - 4, n_b):
            stor_cp(i, i % _NSLOT).wait()

    return k


def kernel(x, table):
    B, S = x.shape
    V, D = table.shape
    info = plsc.get_sparse_core_info()
    untile, Vp = _tc_untile(V, D)
    t_lin = untile(table.T).reshape(Vp, D)
    xi = x.astype(jnp.int32)
    hbits = _H.bit_length() - 1
    xi = (xi & ~(_BV - 1)) | ((xi & (_H - 1)) << 1) | ((xi >> hbits) & 1)
    out = _sc_embed(B, S, D, info.num_cores, info.num_subcores)(t_lin, xi)
    return out[:, :D].reshape(B, S, D)


# speedup vs baseline: 2.2201x; 1.0586x over previous
import functools

import jax
import jax.numpy as jnp
from jax import lax
from jax.experimental import pallas as pl
from jax.experimental.pallas import tpu as pltpu
from jax.experimental.pallas import tpu_sc as plsc

_NSLOT = 4
_BV = 16384
_H = _BV // 2


def _tc_untile(V, D):
    grid = pl.cdiv(V, _BV)
    Vp = grid * _BV

    def body(i_ref, o_ref):
        x1 = i_ref[:, :_H].T
        x2 = i_ref[:, _H:].T
        o_ref[...] = jnp.concatenate([x1, x2], axis=1)

    return pl.pallas_call(
        body,
        grid=(grid,),
        in_specs=[pl.BlockSpec((D, _BV), lambda j: (0, j))],
        out_specs=pl.BlockSpec((_BV // 2, 2 * D), lambda j: (j, 0)),
        out_shape=jax.ShapeDtypeStruct((Vp * D // 128, 128), jnp.float32),
    ), Vp


def _sc_embed(B, S, D, num_cores, num_subcores):
    NW = num_cores * num_subcores
    n_b = B // NW
    SP = 256
    mesh = plsc.VectorSubcoreMesh(core_axis_name="c", subcore_axis_name="s")

    @functools.partial(
        pl.kernel,
        mesh=mesh,
        out_type=jax.ShapeDtypeStruct((B * S, 2 * D), jnp.float32),
        scratch_types=[
            pltpu.VMEM((_NSLOT * SP,), jnp.int32),
            pltpu.VMEM((_NSLOT, S, D), jnp.float32),
        ]
        + [pltpu.SemaphoreType.DMA] * (3 * _NSLOT),
        compiler_params=pltpu.CompilerParams(use_tc_tiling_on_sc=False),
    )
    def k(table_hbm, x_hbm, out_hbm, idx_v, rows_v, *sems):
        isem = sems[0:_NSLOT]
        gsem = sems[_NSLOT:2 * _NSLOT]
        ssem = sems[2 * _NSLOT:3 * _NSLOT]
        wid = lax.axis_index("s") * num_cores + lax.axis_index("c")
        b0 = wid * n_b

        def idx_cp(i, r):
            return pltpu.make_async_copy(
                x_hbm.at[b0 + i], idx_v.at[pl.ds(r * SP, S)], isem[r])

        def gath_cp(i, r):
            return pltpu.make_async_copy(
                table_hbm.at[idx_v.at[pl.ds(r * SP, S)]], rows_v.at[r],
                gsem[r])

        def stor_cp(i, r):
            return pltpu.make_async_copy(
                rows_v.at[r],
                out_hbm.at[pl.ds((b0 + i) * S, S), pl.ds(0, D)], ssem[r])

        def body(i, r):
            stor_cp(i - 2, (r - 2) % _NSLOT).wait()
            idx_cp(i + 2, (r + 2) % _NSLOT).wait()
            gath_cp(i + 2, (r + 2) % _NSLOT).start()
            gath_cp(i, r).wait()
            stor_cp(i, r).start()
            idx_cp(i + 4, r).start()

        for i in range(_NSLOT):
            idx_cp(i, i).start()
        for i in range(2):
            idx_cp(i, i).wait()
            gath_cp(i, i).start()
        idx_cp(2, 2).wait()
        gath_cp(2, 2).start()
        gath_cp(0, 0).wait()
        stor_cp(0, 0).start()
        idx_cp(4, 0).start()
        idx_cp(3, 3).wait()
        gath_cp(3, 3).start()
        gath_cp(1, 1).wait()
        stor_cp(1, 1).start()
        idx_cp(5, 1).start()
        body(2, 2)
        body(3, 3)

        def outer(kk, carry):
            i4 = 4 * kk
            for r in range(_NSLOT):
                body(i4 + r, r)
            return carry

        lax.fori_loop(1, (n_b - 4) // 4, outer, 0)

        i = n_b - 4
        stor_cp(i - 2, (i - 2) % _NSLOT).wait()
        idx_cp(i + 2, (i + 2) % _NSLOT).wait()
        gath_cp(i + 2, (i + 2) % _NSLOT).start()
        gath_cp(i, i % _NSLOT).wait()
        stor_cp(i, i % _NSLOT).start()
        i = n_b - 3
        stor_cp(i - 2, (i - 2) % _NSLOT).wait()
        idx_cp(i + 2, (i + 2) % _NSLOT).wait()
        gath_cp(i + 2, (i + 2) % _NSLOT).start()
        gath_cp(i, i % _NSLOT).wait()
        stor_cp(i, i % _NSLOT).start()
        for i in range(n_b - 2, n_b):
            gath_cp(i, i % _NSLOT).wait()
            stor_cp(i, i % _NSLOT).start()
        for i in range(n_b - 4, n_b):
            stor_cp(i, i % _NSLOT).wait()

    return k


def kernel(x, table):
    B, S = x.shape
    V, D = table.shape
    info = plsc.get_sparse_core_info()
    untile, Vp = _tc_untile(V, D)
    t_lin = untile(table.T).reshape(Vp, D)
    xi = x.astype(jnp.int32)
    hbits = _H.bit_length() - 1
    xi = (xi & ~(_BV - 1)) | ((xi & (_H - 1)) << 1) | ((xi >> hbits) & 1)
    out = _sc_embed(B, S, D, info.num_cores, info.num_subcores)(t_lin, xi)
    return out[:, :D].reshape(B, S, D)


# speedup vs baseline: 2.2751x; 1.0248x over previous
import functools

import jax
import jax.numpy as jnp
from jax import lax
from jax.experimental import pallas as pl
from jax.experimental.pallas import tpu as pltpu
from jax.experimental.pallas import tpu_sc as plsc

_NSLOT = 4
_BV = 32768
_H = _BV // 2


def _tc_untile(V, D):
    grid = pl.cdiv(V, _BV)
    Vp = grid * _BV

    def body(i_ref, o_ref):
        x1 = i_ref[:, :_H].T
        x2 = i_ref[:, _H:].T
        o_ref[...] = jnp.concatenate([x1, x2], axis=1)

    return pl.pallas_call(
        body,
        grid=(grid,),
        in_specs=[pl.BlockSpec((D, _BV), lambda j: (0, j))],
        out_specs=pl.BlockSpec((_BV // 2, 2 * D), lambda j: (j, 0)),
        out_shape=jax.ShapeDtypeStruct((Vp * D // 128, 128), jnp.float32),
    ), Vp


def _sc_embed(B, S, D, num_cores, num_subcores):
    NW = num_cores * num_subcores
    n_b = B // NW
    SP = 256
    mesh = plsc.VectorSubcoreMesh(core_axis_name="c", subcore_axis_name="s")

    @functools.partial(
        pl.kernel,
        mesh=mesh,
        out_type=jax.ShapeDtypeStruct((B * S, 2 * D), jnp.float32),
        scratch_types=[
            pltpu.VMEM((_NSLOT * SP,), jnp.int32),
            pltpu.VMEM((_NSLOT, S, D), jnp.float32),
        ]
        + [pltpu.SemaphoreType.DMA] * (3 * _NSLOT),
        compiler_params=pltpu.CompilerParams(use_tc_tiling_on_sc=False),
    )
    def k(table_hbm, x_hbm, out_hbm, idx_v, rows_v, *sems):
        isem = sems[0:_NSLOT]
        gsem = sems[_NSLOT:2 * _NSLOT]
        ssem = sems[2 * _NSLOT:3 * _NSLOT]
        wid = lax.axis_index("s") * num_cores + lax.axis_index("c")
        b0 = wid * n_b

        def idx_cp(i, r):
            return pltpu.make_async_copy(
                x_hbm.at[b0 + i], idx_v.at[pl.ds(r * SP, S)], isem[r])

        def gath_cp(i, r):
            return pltpu.make_async_copy(
                table_hbm.at[idx_v.at[pl.ds(r * SP, S)]], rows_v.at[r],
                gsem[r])

        def stor_cp(i, r):
            return pltpu.make_async_copy(
                rows_v.at[r],
                out_hbm.at[pl.ds((b0 + i) * S, S), pl.ds(0, D)], ssem[r])

        def body(i, r):
            stor_cp(i - 2, (r - 2) % _NSLOT).wait()
            idx_cp(i + 2, (r + 2) % _NSLOT).wait()
            gath_cp(i + 2, (r + 2) % _NSLOT).start()
            gath_cp(i, r).wait()
            stor_cp(i, r).start()
            idx_cp(i + 4, r).start()

        for i in range(_NSLOT):
            idx_cp(i, i).start()
        for i in range(2):
            idx_cp(i, i).wait()
            gath_cp(i, i).start()
        idx_cp(2, 2).wait()
        gath_cp(2, 2).start()
        gath_cp(0, 0).wait()
        stor_cp(0, 0).start()
        idx_cp(4, 0).start()
        idx_cp(3, 3).wait()
        gath_cp(3, 3).start()
        gath_cp(1, 1).wait()
        stor_cp(1, 1).start()
        idx_cp(5, 1).start()
        body(2, 2)
        body(3, 3)

        def outer(kk, carry):
            i4 = 4 * kk
            for r in range(_NSLOT):
                body(i4 + r, r)
            return carry

        lax.fori_loop(1, (n_b - 4) // 4, outer, 0)

        i = n_b - 4
        stor_cp(i - 2, (i - 2) % _NSLOT).wait()
        idx_cp(i + 2, (i + 2) % _NSLOT).wait()
        gath_cp(i + 2, (i + 2) % _NSLOT).start()
        gath_cp(i, i % _NSLOT).wait()
        stor_cp(i, i % _NSLOT).start()
        i = n_b - 3
        stor_cp(i - 2, (i - 2) % _NSLOT).wait()
        idx_cp(i + 2, (i + 2) % _NSLOT).wait()
        gath_cp(i + 2, (i + 2) % _NSLOT).start()
        gath_cp(i, i % _NSLOT).wait()
        stor_cp(i, i % _NSLOT).start()
        for i in range(n_b - 2, n_b):
            gath_cp(i, i % _NSLOT).wait()
            stor_cp(i, i % _NSLOT).start()
        for i in range(n_b - 4, n_b):
            stor_cp(i, i % _NSLOT).wait()

    return k


def kernel(x, table):
    B, S = x.shape
    V, D = table.shape
    info = plsc.get_sparse_core_info()
    untile, Vp = _tc_untile(V, D)
    t_lin = untile(table.T).reshape(Vp, D)
    xi = x.astype(jnp.int32)
    hbits = _H.bit_length() - 1
    xi = (xi & ~(_BV - 1)) | ((xi & (_H - 1)) << 1) | ((xi >> hbits) & 1)
    out = _sc_embed(B, S, D, info.num_cores, info.num_subcores)(t_lin, xi)
    return out[:, :D].reshape(B, S, D)
